# baseline (speedup 1.0000x reference)
import functools

import jax
import jax.numpy as jnp
from jax import lax
from jax.experimental import pallas as pl
from jax.experimental.pallas import tpu as pltpu
from jax.experimental.pallas import tpu_sc as plsc

_NC, _NS = 2, 16
_NW = _NC * _NS


def _accum3(acc, b, c, nrows, ncol):
    def body(r, carry):
        for g in range(ncol // 16):
            sl = pl.ds(g * 16, 16)
            acc[r, sl] = acc[r, sl] + b[r, sl] + c[r, sl]
        return carry
    lax.fori_loop(0, nrows, body, 0)


def _reduce4(dst, src, nrows_out, ncol):
    def body(t, carry):
        for g in range(ncol // 16):
            sl = pl.ds(g * 16, 16)
            dst[t, sl] = (src[4 * t, sl] + src[4 * t + 1, sl]
                          + src[4 * t + 2, sl] + src[4 * t + 3, sl])
        return carry
    lax.fori_loop(0, nrows_out, body, 0)


def _mesh():
    return plsc.VectorSubcoreMesh(core_axis_name="c", subcore_axis_name="s",
                                  num_cores=_NC, num_subcores=_NS)


def _sc_pre_stage(s_tok, i_tok, a_tok, m_flat,
                  pre_species, pre_items, pre_abilities, pre_moves):
    B = s_tok.shape[0]
    P = pre_species.shape[1]
    f32 = jnp.float32
    SUB = 64
    chunk = B // _NW
    nstep = chunk // SUB

    @functools.partial(
        pl.kernel,
        out_type=(
            jax.ShapeDtypeStruct((B, P), f32),
            jax.ShapeDtypeStruct((B, P), f32),
            jax.ShapeDtypeStruct((B, P), f32),
            jax.ShapeDtypeStruct((B, P), f32),
        ),
        mesh=_mesh(),
        compiler_params=pltpu.CompilerParams(use_tc_tiling_on_sc=True),
        scratch_types=[
            pltpu.VMEM((SUB,), jnp.int32),
            pltpu.VMEM((SUB,), jnp.int32),
            pltpu.VMEM((SUB,), jnp.int32),
            pltpu.VMEM((2, 2 * SUB), jnp.int32),
            pltpu.VMEM((SUB, P), f32),
            pltpu.VMEM((SUB, P), f32),
            pltpu.VMEM((SUB, P), f32),
            pltpu.VMEM((4 * SUB, P), f32),
            pltpu.VMEM((SUB, P), f32),
            pltpu.SemaphoreType.DMA,
            pltpu.SemaphoreType.DMA,
            pltpu.SemaphoreType.DMA,
            pltpu.SemaphoreType.DMA,
        ],
    )
    def sc_k(s_hbm, i_hbm, a_hbm, m_hbm,
             ps_hbm, pi_hbm, pa_hbm, pm_hbm,
             preS_hbm, preI_hbm, preA_hbm, pmsum_hbm,
             sidx, iidx, aidx, midx, prebufS, prebufI, prebufA,
             pmrows, pmsumb, semS, semI, semA, semM):
        wid = lax.axis_index("s") * _NC + lax.axis_index("c")
        tile_base = wid * chunk

        def step_body(step, carry):
            base = tile_base + step * SUB
            pltpu.sync_copy(s_hbm.at[pl.ds(base, SUB)], sidx)
            pltpu.sync_copy(i_hbm.at[pl.ds(base, SUB)], iidx)
            pltpu.sync_copy(a_hbm.at[pl.ds(base, SUB)], aidx)
            pltpu.sync_copy(m_hbm.at[pl.ds(4 * base, 2 * SUB)], midx.at[0])
            pltpu.sync_copy(m_hbm.at[pl.ds(4 * base + 2 * SUB, 2 * SUB)],
                            midx.at[1])

            cPS = pltpu.async_copy(ps_hbm.at[sidx], prebufS, semS)
            cPI = pltpu.async_copy(pi_hbm.at[iidx], prebufI, semI)
            cPA = pltpu.async_copy(pa_hbm.at[aidx], prebufA, semA)
            cPM0 = pltpu.async_copy(pm_hbm.at[midx.at[0]],
                                    pmrows.at[pl.ds(0, 2 * SUB)], semM)
            cPM1 = pltpu.async_copy(pm_hbm.at[midx.at[1]],
                                    pmrows.at[pl.ds(2 * SUB, 2 * SUB)], semM)

            cPS.wait()
            pltpu.sync_copy(prebufS, preS_hbm.at[pl.ds(base, SUB)])
            cPI.wait()
            pltpu.sync_copy(prebufI, preI_hbm.at[pl.ds(base, SUB)])
            cPA.wait()
            pltpu.sync_copy(prebufA, preA_hbm.at[pl.ds(base, SUB)])
            cPM0.wait()
            cPM1.wait()
            _reduce4(pmsumb, pmrows, SUB, P)
            pltpu.sync_copy(pmsumb, pmsum_hbm.at[pl.ds(base, SUB)])
            return carry

        lax.fori_loop(0, nstep, step_body, 0)

    return sc_k(s_tok, i_tok, a_tok, m_flat,
                pre_species, pre_items, pre_abilities, pre_moves)


def _sc_learned_stage(s_tok, i_tok, a_tok, m_flat,
                      species_table, items_table, abilities_table,
                      moves_table):
    B = s_tok.shape[0]
    D = species_table.shape[1]
    f32 = jnp.float32
    SUB = 128
    chunk = B // _NW
    nstep = chunk // SUB

    @functools.partial(
        pl.kernel,
        out_type=(
            jax.ShapeDtypeStruct((B, D), f32),
            jax.ShapeDtypeStruct((B, D), f32),
        ),
        mesh=_mesh(),
        compiler_params=pltpu.CompilerParams(use_tc_tiling_on_sc=False),
        scratch_types=[
            pltpu.VMEM((SUB,), jnp.int32),
            pltpu.VMEM((SUB,), jnp.int32),
            pltpu.VMEM((SUB,), jnp.int32),
            pltpu.VMEM((4, SUB), jnp.int32),
            pltpu.VMEM((SUB, D), f32),
            pltpu.VMEM((SUB, D), f32),
            pltpu.VMEM((SUB, D), f32),
            pltpu.VMEM((4 * SUB, D), f32),
            pltpu.VMEM((SUB, D), f32),
            pltpu.SemaphoreType.DMA,
            pltpu.SemaphoreType.DMA,
            pltpu.SemaphoreType.DMA,
            pltpu.SemaphoreType.DMA,
        ],
    )
    def sc_k(s_hbm, i_hbm, a_hbm, m_hbm,
             st_hbm, it_hbm, at_hbm, mt_hbm,
             embsum_hbm, msum_hbm,
             sidx, iidx, aidx, midx, rowsS, rowsI, rowsA, mrows, msumb,
             semS, semI, semA, semM):
        wid = lax.axis_index("s") * _NC + lax.axis_index("c")
        tile_base = wid * chunk

        def step_body(step, carry):
            base = tile_base + step * SUB
            pltpu.sync_copy(s_hbm.at[pl.ds(base, SUB)], sidx)
            pltpu.sync_copy(i_hbm.at[pl.ds(base, SUB)], iidx)
            pltpu.sync_copy(a_hbm.at[pl.ds(base, SUB)], aidx)
            for j in range(4):
                pltpu.sync_copy(m_hbm.at[pl.ds(4 * base + j * SUB, SUB)],
                                midx.at[j])

            cS = pltpu.async_copy(st_hbm.at[sidx], rowsS, semS)
            cI = pltpu.async_copy(it_hbm.at[iidx], rowsI, semI)
            cA = pltpu.async_copy(at_hbm.at[aidx], rowsA, semA)
            cM = [pltpu.async_copy(mt_hbm.at[midx.at[j]],
                                   mrows.at[pl.ds(j * SUB, SUB)], semM)
                  for j in range(4)]

            cS.wait()
            cI.wait()
            cA.wait()
            _accum3(rowsS, rowsI, rowsA, SUB, D)
            pltpu.sync_copy(rowsS, embsum_hbm.at[pl.ds(base, SUB)])

            for c in cM:
                c.wait()
            _reduce4(msumb, mrows, SUB, D)
            pltpu.sync_copy(msumb, msum_hbm.at[pl.ds(base, SUB)])
            return carry

        lax.fori_loop(0, nstep, step_body, 0)

    return sc_k(s_tok, i_tok, a_tok, m_flat,
                species_table, items_table, abilities_table, moves_table)


def _tc_combine(embsum, msum, preS, preI, preA, pmsum,
                s_tok2, i_tok2, a_tok2, m_tok, nm2,
                Ws, Wi, Wa, Wm, r0s, r0i, r0a, r0m):
    B, D = embsum.shape
    P = preS.shape[1]
    BLK = min(1024, B)
    f32 = jnp.float32

    def body(emb_r, msum_r, ps_r, pi_r, pa_r, pm_r,
             st_r, it_r, at_r, mt_r, nm_r,
             ws_r, wi_r, wa_r, wm_r, r0s_r, r0i_r, r0a_r, r0m_r, out_r):
        ws = ws_r[...]
        wi = wi_r[...]
        wa = wa_r[...]
        wm = wm_r[...]
        cs = jnp.dot(r0s_r[...], ws, preferred_element_type=f32)
        ci = jnp.dot(r0i_r[...], wi, preferred_element_type=f32)
        ca = jnp.dot(r0a_r[...], wa, preferred_element_type=f32)
        cm = jnp.dot(r0m_r[...], wm, preferred_element_type=f32)
        zs = (st_r[...] == 0).astype(f32)
        zi = (it_r[...] == 0).astype(f32)
        za = (at_r[...] == 0).astype(f32)
        cnt0 = jnp.sum((mt_r[...] == 0).astype(f32), axis=1, keepdims=True)
        nmf = jnp.maximum(nm_r[...], 1).astype(f32)
        lin = (jnp.dot(ps_r[...], ws, preferred_element_type=f32) - zs * cs
               + jnp.dot(pi_r[...], wi, preferred_element_type=f32) - zi * ci
               + jnp.dot(pa_r[...], wa, preferred_element_type=f32) - za * ca)
        mv = (jnp.dot(pm_r[...], wm, preferred_element_type=f32) - cnt0 * cm)
        out_r[...] = emb_r[...] + (msum_r[...] + mv) / nmf + lin

    blk_bd = pl.BlockSpec((BLK, D), lambda i: (i, 0))
    blk_bp = pl.BlockSpec((BLK, P), lambda i: (i, 0))
    blk_b1 = pl.BlockSpec((BLK, 1), lambda i: (i, 0))
    blk_b4 = pl.BlockSpec((BLK, 4), lambda i: (i, 0))
    blk_w = pl.BlockSpec((P, D), lambda i: (0, 0))
    blk_r0 = pl.BlockSpec((1, P), lambda i: (0, 0))

    return pl.pallas_call(
        body,
        grid=(B // BLK,),
        in_specs=[blk_bd, blk_bd, blk_bp, blk_bp, blk_bp, blk_bp,
                  blk_b1, blk_b1, blk_b1, blk_b4, blk_b1,
                  blk_w, blk_w, blk_w, blk_w,
                  blk_r0, blk_r0, blk_r0, blk_r0],
        out_specs=blk_bd,
        out_shape=jax.ShapeDtypeStruct((B, D), f32),
    )(embsum, msum, preS, preI, preA, pmsum,
      s_tok2, i_tok2, a_tok2, m_tok, nm2,
      Ws, Wi, Wa, Wm, r0s, r0i, r0a, r0m)


def kernel(species_tokens, ability_tokens, item_tokens, move_tokens, num_moves,
           species_table, items_table, abilities_table, moves_table,
           pre_species, pre_items, pre_abilities, pre_moves,
           species_W, items_W, abilities_W, moves_W):
    m_flat = move_tokens.reshape(-1)
    preS, preI, preA, pmsum = _sc_pre_stage(
        species_tokens, item_tokens, ability_tokens, m_flat,
        pre_species, pre_items, pre_abilities, pre_moves)
    embsum, msum = _sc_learned_stage(
        species_tokens, item_tokens, ability_tokens, m_flat,
        species_table, items_table, abilities_table, moves_table)
    return _tc_combine(
        embsum, msum, preS, preI, preA, pmsum,
        species_tokens[:, None], item_tokens[:, None], ability_tokens[:, None],
        move_tokens, num_moves[:, None],
        species_W, items_W, abilities_W, moves_W,
        pre_species[0:1], pre_items[0:1], pre_abilities[0:1], pre_moves[0:1])


# speedup vs baseline: 1.3504x; 1.3504x over previous
import functools

import jax
import jax.numpy as jnp
from jax import lax
from jax.experimental import pallas as pl
from jax.experimental.pallas import tpu as pltpu
from jax.experimental.pallas import tpu_sc as plsc

_NC, _NS = 2, 16
_NW = _NC * _NS


def _tc_fold(lTs, lTi, lTa, lTm, preS, preI, preA, preM, Ws, Wi, Wa, Wm):
    D, V = lTs.shape
    P = preS.shape[1]
    VB = 1024
    grid = ((V + VB - 1) // VB,)
    f32 = jnp.float32

    def body(lts_r, lti_r, lta_r, ltm_r, ps_r, pi_r, pa_r, pm_r,
             ws_r, wi_r, wa_r, wm_r, x_r, y_r):
        def fused(lt_r, p_r, w_r):
            lt = jnp.transpose(lt_r[...], (1, 0))
            return lt + jnp.dot(p_r[...], w_r[...], preferred_element_type=f32)
        s = fused(lts_r, ps_r, ws_r)
        i = fused(lti_r, pi_r, wi_r)
        a = fused(lta_r, pa_r, wa_r)
        m = fused(ltm_r, pm_r, wm_r)
        x_r[...] = jnp.concatenate([s, i], axis=1)
        y_r[...] = jnp.concatenate([a, m], axis=1)

    blk_lt = pl.BlockSpec((D, VB), lambda i: (0, i))
    blk_p = pl.BlockSpec((VB, P), lambda i: (i, 0))
    blk_w = pl.BlockSpec((P, D), lambda i: (0, 0))
    blk_o = pl.BlockSpec((VB, 2 * D), lambda i: (i, 0))

    return pl.pallas_call(
        body,
        grid=grid,
        in_specs=[blk_lt, blk_lt, blk_lt, blk_lt,
                  blk_p, blk_p, blk_p, blk_p,
                  blk_w, blk_w, blk_w, blk_w],
        out_specs=[blk_o, blk_o],
        out_shape=[jax.ShapeDtypeStruct((V, 2 * D), f32),
                   jax.ShapeDtypeStruct((V, 2 * D), f32)],
    )(lTs, lTi, lTa, lTm, preS, preI, preA, preM, Ws, Wi, Wa, Wm)


def _sc_gather(s_tok, i_tok, a_tok, m_flat, X, Y):
    B = s_tok.shape[0]
    W2 = X.shape[1]
    D = W2 // 2
    f32 = jnp.float32
    SUB = 64
    chunk = B // _NW
    nstep = chunk // SUB

    mesh = plsc.VectorSubcoreMesh(core_axis_name="c", subcore_axis_name="s",
                                  num_cores=_NC, num_subcores=_NS)

    @functools.partial(
        pl.kernel,
        out_type=(
            jax.ShapeDtypeStruct((B, D), f32),
            jax.ShapeDtypeStruct((B, D), f32),
        ),
        mesh=mesh,
        compiler_params=pltpu.CompilerParams(use_tc_tiling_on_sc=True),
        scratch_types=[
            pltpu.VMEM((SUB,), jnp.int32),
            pltpu.VMEM((SUB,), jnp.int32),
            pltpu.VMEM((SUB,), jnp.int32),
            pltpu.VMEM((2, 2 * SUB), jnp.int32),
            pltpu.VMEM((SUB, W2), f32),
            pltpu.VMEM((SUB, W2), f32),
            pltpu.VMEM((SUB, W2), f32),
            pltpu.VMEM((4 * SUB, W2), f32),
            pltpu.VMEM((SUB, D), f32),
            pltpu.VMEM((SUB, D), f32),
            pltpu.SemaphoreType.DMA,
            pltpu.SemaphoreType.DMA,
            pltpu.SemaphoreType.DMA,
            pltpu.SemaphoreType.DMA,
        ],
    )
    def sc_k(s_hbm, i_hbm, a_hbm, m_hbm, x_hbm, y_hbm,
             embsum_hbm, msum_hbm,
             sidx, iidx, aidx, midx, bufS, bufI, bufA, mrows, accb, msumb,
             semS, semI, semA, semM):
        wid = lax.axis_index("s") * _NC + lax.axis_index("c")
        tile_base = wid * chunk

        def step_body(step, carry):
            base = tile_base + step * SUB
            pltpu.sync_copy(s_hbm.at[pl.ds(base, SUB)], sidx)
            pltpu.sync_copy(i_hbm.at[pl.ds(base, SUB)], iidx)
            pltpu.sync_copy(a_hbm.at[pl.ds(base, SUB)], aidx)
            pltpu.sync_copy(m_hbm.at[pl.ds(4 * base, 2 * SUB)], midx.at[0])
            pltpu.sync_copy(m_hbm.at[pl.ds(4 * base + 2 * SUB, 2 * SUB)],
                            midx.at[1])

            cS = pltpu.async_copy(x_hbm.at[sidx], bufS, semS)
            cI = pltpu.async_copy(x_hbm.at[iidx], bufI, semI)
            cA = pltpu.async_copy(y_hbm.at[aidx], bufA, semA)
            cM0 = pltpu.async_copy(y_hbm.at[midx.at[0]],
                                   mrows.at[pl.ds(0, 2 * SUB)], semM)
            cM1 = pltpu.async_copy(y_hbm.at[midx.at[1]],
                                   mrows.at[pl.ds(2 * SUB, 2 * SUB)], semM)

            cS.wait()
            cI.wait()
            cA.wait()

            def acc_body(r, carry2):
                for g in range(D // 16):
                    sl = pl.ds(g * 16, 16)
                    sl_hi = pl.ds(D + g * 16, 16)
                    accb[r, sl] = bufS[r, sl] + bufI[r, sl_hi] + bufA[r, sl]
                return carry2
            lax.fori_loop(0, SUB, acc_body, 0)
            pltpu.sync_copy(accb, embsum_hbm.at[pl.ds(base, SUB)])

            cM0.wait()
            cM1.wait()

            def msum_body(t, carry2):
                for g in range(D // 16):
                    sl = pl.ds(g * 16, 16)
                    sl_hi = pl.ds(D + g * 16, 16)
                    msumb[t, sl] = (mrows[4 * t, sl_hi]
                                    + mrows[4 * t + 1, sl_hi]
                                    + mrows[4 * t + 2, sl_hi]
                                    + mrows[4 * t + 3, sl_hi])
                return carry2
            lax.fori_loop(0, SUB, msum_body, 0)
            pltpu.sync_copy(msumb, msum_hbm.at[pl.ds(base, SUB)])
            return carry

        lax.fori_loop(0, nstep, step_body, 0)

    return sc_k(s_tok, i_tok, a_tok, m_flat, X, Y)


def _tc_combine(embsum, msum, s_tok2, i_tok2, a_tok2, m_tok, nm2,
                Ws, Wi, Wa, Wm, r0s, r0i, r0a, r0m):
    B, D = embsum.shape
    P = r0s.shape[1]
    BLK = min(2048, B)
    f32 = jnp.float32

    def body(emb_r, msum_r, st_r, it_r, at_r, mt_r, nm_r,
             ws_r, wi_r, wa_r, wm_r, r0s_r, r0i_r, r0a_r, r0m_r, out_r):
        cs = jnp.dot(r0s_r[...], ws_r[...], preferred_element_type=f32)
        ci = jnp.dot(r0i_r[...], wi_r[...], preferred_element_type=f32)
        ca = jnp.dot(r0a_r[...], wa_r[...], preferred_element_type=f32)
        cm = jnp.dot(r0m_r[...], wm_r[...], preferred_element_type=f32)
        zs = (st_r[...] == 0).astype(f32)
        zi = (it_r[...] == 0).astype(f32)
        za = (at_r[...] == 0).astype(f32)
        cnt0 = jnp.sum((mt_r[...] == 0).astype(f32), axis=1, keepdims=True)
        nmf = jnp.maximum(nm_r[...], 1).astype(f32)
        out_r[...] = (emb_r[...] + (msum_r[...] - cnt0 * cm) / nmf
                      - zs * cs - zi * ci - za * ca)

    blk_bd = pl.BlockSpec((BLK, D), lambda i: (i, 0))
    blk_b1 = pl.BlockSpec((BLK, 1), lambda i: (i, 0))
    blk_b4 = pl.BlockSpec((BLK, 4), lambda i: (i, 0))
    blk_w = pl.BlockSpec((P, D), lambda i: (0, 0))
    blk_r0 = pl.BlockSpec((1, P), lambda i: (0, 0))

    return pl.pallas_call(
        body,
        grid=(B // BLK,),
        in_specs=[blk_bd, blk_bd,
                  blk_b1, blk_b1, blk_b1, blk_b4, blk_b1,
                  blk_w, blk_w, blk_w, blk_w,
                  blk_r0, blk_r0, blk_r0, blk_r0],
        out_specs=blk_bd,
        out_shape=jax.ShapeDtypeStruct((B, D), f32),
    )(embsum, msum, s_tok2, i_tok2, a_tok2, m_tok, nm2,
      Ws, Wi, Wa, Wm, r0s, r0i, r0a, r0m)


def kernel(species_tokens, ability_tokens, item_tokens, move_tokens, num_moves,
           species_table, items_table, abilities_table, moves_table,
           pre_species, pre_items, pre_abilities, pre_moves,
           species_W, items_W, abilities_W, moves_W):
    m_flat = move_tokens.reshape(-1)
    X, Y = _tc_fold(
        species_table.T, items_table.T, abilities_table.T, moves_table.T,
        pre_species, pre_items, pre_abilities, pre_moves,
        species_W, items_W, abilities_W, moves_W)
    embsum, msum = _sc_gather(
        species_tokens, item_tokens, ability_tokens, m_flat, X, Y)
    return _tc_combine(
        embsum, msum,
        species_tokens[:, None], item_tokens[:, None], ability_tokens[:, None],
        move_tokens, num_moves[:, None],
        species_W, items_W, abilities_W, moves_W,
        pre_species[0:1], pre_items[0:1], pre_abilities[0:1], pre_moves[0:1])


# speedup vs baseline: 1.5933x; 1.1799x over previous
import functools

import jax
import jax.numpy as jnp
from jax import lax
from jax.experimental import pallas as pl
from jax.experimental.pallas import tpu as pltpu
from jax.experimental.pallas import tpu_sc as plsc

_NC, _NS = 2, 16
_NW = _NC * _NS


def _tc_fold_pair(lt1, lt2, pre1, pre2, W1, W2):
    D, V = lt1.shape
    P = pre1.shape[1]
    VB = 2048
    grid = ((V + VB - 1) // VB,)
    f32 = jnp.float32

    def body(lt1_r, lt2_r, p1_r, p2_r, w1_r, w2_r, out_r):
        def fused(lt_r, p_r, w_r):
            lt = jnp.transpose(lt_r[...], (1, 0))
            return lt + jnp.dot(p_r[...], w_r[...], preferred_element_type=f32)
        out_r[...] = jnp.concatenate(
            [fused(lt1_r, p1_r, w1_r), fused(lt2_r, p2_r, w2_r)], axis=1)

    blk_lt = pl.BlockSpec((D, VB), lambda i: (0, i))
    blk_p = pl.BlockSpec((VB, P), lambda i: (i, 0))
    blk_w = pl.BlockSpec((P, D), lambda i: (0, 0))
    blk_o = pl.BlockSpec((VB, 2 * D), lambda i: (i, 0))

    return pl.pallas_call(
        body,
        grid=grid,
        in_specs=[blk_lt, blk_lt, blk_p, blk_p, blk_w, blk_w],
        out_specs=blk_o,
        out_shape=jax.ShapeDtypeStruct((V, 2 * D), f32),
    )(lt1, lt2, pre1, pre2, W1, W2)


def _mesh():
    return plsc.VectorSubcoreMesh(core_axis_name="c", subcore_axis_name="s",
                                  num_cores=_NC, num_subcores=_NS)


def _sc_gather_x(s_tok, i_tok, X):
    B = s_tok.shape[0]
    W2 = X.shape[1]
    D = W2 // 2
    f32 = jnp.float32
    SUB = 128
    chunk = B // _NW
    nstep = chunk // SUB

    @functools.partial(
        pl.kernel,
        out_type=jax.ShapeDtypeStruct((B, D), f32),
        mesh=_mesh(),
        compiler_params=pltpu.CompilerParams(use_tc_tiling_on_sc=True),
        scratch_types=[
            pltpu.VMEM((SUB,), jnp.int32),
            pltpu.VMEM((SUB,), jnp.int32),
            pltpu.VMEM((SUB, W2), f32),
            pltpu.VMEM((SUB, W2), f32),
            pltpu.VMEM((SUB, D), f32),
            pltpu.SemaphoreType.DMA,
            pltpu.SemaphoreType.DMA,
        ],
    )
    def sc_k(s_hbm, i_hbm, x_hbm, emb_hbm,
             sidx, iidx, bufS, bufI, accb, semS, semI):
        wid = lax.axis_index("s") * _NC + lax.axis_index("c")
        tile_base = wid * chunk

        def step_body(step, carry):
            base = tile_base + step * SUB
            pltpu.sync_copy(s_hbm.at[pl.ds(base, SUB)], sidx)
            pltpu.sync_copy(i_hbm.at[pl.ds(base, SUB)], iidx)
            cS = pltpu.async_copy(x_hbm.at[sidx], bufS, semS)
            cI = pltpu.async_copy(x_hbm.at[iidx], bufI, semI)
            cS.wait()
            cI.wait()

            def acc_body(r, carry2):
                for g in range(D // 16):
                    sl = pl.ds(g * 16, 16)
                    sl_hi = pl.ds(D + g * 16, 16)
                    accb[r, sl] = bufS[r, sl] + bufI[r, sl_hi]
                return carry2
            lax.fori_loop(0, SUB, acc_body, 0)
            pltpu.sync_copy(accb, emb_hbm.at[pl.ds(base, SUB)])
            return carry

        lax.fori_loop(0, nstep, step_body, 0)

    return sc_k(s_tok, i_tok, X)


def _sc_gather_y(a_tok, m_flat, Y):
    B = a_tok.shape[0]
    W2 = Y.shape[1]
    D = W2 // 2
    f32 = jnp.float32
    SUB = 128
    chunk = B // _NW
    nstep = chunk // SUB

    @functools.partial(
        pl.kernel,
        out_type=(
            jax.ShapeDtypeStruct((B, D), f32),
            jax.ShapeDtypeStruct((B, D), f32),
        ),
        mesh=_mesh(),
        compiler_params=pltpu.CompilerParams(use_tc_tiling_on_sc=True),
        scratch_types=[
            pltpu.VMEM((SUB,), jnp.int32),
            pltpu.VMEM((4, SUB), jnp.int32),
            pltpu.VMEM((SUB, W2), f32),
            pltpu.VMEM((4 * SUB, W2), f32),
            pltpu.VMEM((SUB, D), f32),
            pltpu.VMEM((SUB, D), f32),
            pltpu.SemaphoreType.DMA,
            pltpu.SemaphoreType.DMA,
        ],
    )
    def sc_k(a_hbm, m_hbm, y_hbm, embA_hbm, msum_hbm,
             aidx, midx, bufA, mrows, accb, msumb, semA, semM):
        wid = lax.axis_index("s") * _NC + lax.axis_index("c")
        tile_base = wid * chunk

        def step_body(step, carry):
            base = tile_base + step * SUB
            pltpu.sync_copy(a_hbm.at[pl.ds(base, SUB)], aidx)
            for j in range(4):
                pltpu.sync_copy(m_hbm.at[pl.ds(4 * base + j * SUB, SUB)],
                                midx.at[j])
            cA = pltpu.async_copy(y_hbm.at[aidx], bufA, semA)
            cM = [pltpu.async_copy(y_hbm.at[midx.at[j]],
                                   mrows.at[pl.ds(j * SUB, SUB)], semM)
                  for j in range(4)]
            cA.wait()

            def acc_body(r, carry2):
                for g in range(D // 16):
                    sl = pl.ds(g * 16, 16)
                    accb[r, sl] = bufA[r, sl] + 0.0
                return carry2
            lax.fori_loop(0, SUB, acc_body, 0)
            pltpu.sync_copy(accb, embA_hbm.at[pl.ds(base, SUB)])

            for c in cM:
                c.wait()

            def msum_body(t, carry2):
                for g in range(D // 16):
                    sl = pl.ds(g * 16, 16)
                    sl_hi = pl.ds(D + g * 16, 16)
                    msumb[t, sl] = (mrows[t, sl_hi]
                                    + mrows[SUB + t, sl_hi]
                                    + mrows[2 * SUB + t, sl_hi]
                                    + mrows[3 * SUB + t, sl_hi])
                return carry2
            lax.fori_loop(0, SUB, msum_body, 0)
            pltpu.sync_copy(msumb, msum_hbm.at[pl.ds(base, SUB)])
            return carry

        lax.fori_loop(0, nstep, step_body, 0)

    return sc_k(a_tok, m_flat, Y)


def _tc_combine(embSI, embA, msum, aux8,
                Ws, Wi, Wa, Wm, r0s, r0i, r0a, r0m):
    B, D = embSI.shape
    P = r0s.shape[1]
    BLK = min(2048, B)
    f32 = jnp.float32

    def body(e1_r, e2_r, msum_r, aux_r,
             ws_r, wi_r, wa_r, wm_r, r0s_r, r0i_r, r0a_r, r0m_r, out_r):
        cs = jnp.dot(r0s_r[...], ws_r[...], preferred_element_type=f32)
        ci = jnp.dot(r0i_r[...], wi_r[...], preferred_element_type=f32)
        ca = jnp.dot(r0a_r[...], wa_r[...], preferred_element_type=f32)
        cm = jnp.dot(r0m_r[...], wm_r[...], preferred_element_type=f32)
        aux = aux_r[...]
        zs = (aux[:, 0:1] == 0).astype(f32)
        zi = (aux[:, 1:2] == 0).astype(f32)
        za = (aux[:, 2:3] == 0).astype(f32)
        nmf = jnp.maximum(aux[:, 3:4], 1).astype(f32)
        cnt0 = jnp.sum((aux[:, 4:8] == 0).astype(f32), axis=1, keepdims=True)
        out_r[...] = (e1_r[...] + e2_r[...] + (msum_r[...] - cnt0 * cm) / nmf
                      - zs * cs - zi * ci - za * ca)

    blk_bd = pl.BlockSpec((BLK, D), lambda i: (i, 0))
    blk_b8 = pl.BlockSpec((BLK, 8), lambda i: (i, 0))
    blk_w = pl.BlockSpec((P, D), lambda i: (0, 0))
    blk_r0 = pl.BlockSpec((1, P), lambda i: (0, 0))

    return pl.pallas_call(
        body,
        grid=(B // BLK,),
        in_specs=[blk_bd, blk_bd, blk_bd, blk_b8,
                  blk_w, blk_w, blk_w, blk_w,
                  blk_r0, blk_r0, blk_r0, blk_r0],
        out_specs=blk_bd,
        out_shape=jax.ShapeDtypeStruct((B, D), f32),
    )(embSI, embA, msum, aux8, Ws, Wi, Wa, Wm, r0s, r0i, r0a, r0m)


def kernel(species_tokens, ability_tokens, item_tokens, move_tokens, num_moves,
           species_table, items_table, abilities_table, moves_table,
           pre_species, pre_items, pre_abilities, pre_moves,
           species_W, items_W, abilities_W, moves_W):
    m_flat = move_tokens.reshape(-1)
    aux8 = jnp.concatenate(
        [species_tokens[:, None], item_tokens[:, None],
         ability_tokens[:, None], num_moves[:, None], move_tokens], axis=1)
    X = _tc_fold_pair(species_table.T, items_table.T,
                      pre_species, pre_items, species_W, items_W)
    Y = _tc_fold_pair(abilities_table.T, moves_table.T,
                      pre_abilities, pre_moves, abilities_W, moves_W)
    embSI = _sc_gather_x(species_tokens, item_tokens, X)
    embA, msum = _sc_gather_y(ability_tokens, m_flat, Y)
    return _tc_combine(
        embSI, embA, msum, aux8,
        species_W, items_W, abilities_W, moves_W,
        pre_species[0:1], pre_items[0:1], pre_abilities[0:1], pre_moves[0:1])
